# baseline (speedup 1.0000x reference)
import functools

import jax
import jax.numpy as jnp
from jax import lax
from jax.experimental import pallas as pl
from jax.experimental.pallas import tpu as pltpu
from jax.experimental.pallas import tpu_sc as plsc

IN_DIM = 1024
HIDDEN = 2048
N_EXPERTS = 8
BT = 256
NBLK = (4096 // BT) + N_EXPERTS - 1
NTOK = 4096
NPAD = (NBLK + 1) * BT


def _make_sc_row_gather(V, D, B):
    info = plsc.get_sparse_core_info()
    NC, NS = info.num_cores, info.num_subcores
    NW = NC * NS
    CH = 32
    R = 3
    assert B % (CH * NW) == 0
    b_per_w = B // NW
    n_chunks = b_per_w // CH
    mesh = plsc.VectorSubcoreMesh(core_axis_name="c", subcore_axis_name="s")

    @functools.partial(
        pl.kernel, mesh=mesh,
        out_type=jax.ShapeDtypeStruct((B, D), jnp.float32),
        scratch_types=[
            pltpu.VMEM((b_per_w,), jnp.int32),
            pltpu.VMEM((R, CH, D), jnp.float32),
        ] + [pltpu.SemaphoreType.DMA] * (2 * R),
    )
    def gather_k(table_hbm, idx_hbm, out_hbm, idx_v, rows_v, *sems):
        g_sems, o_sems = sems[:R], sems[R:]
        wid = lax.axis_index("s") * NC + lax.axis_index("c")
        base = wid * b_per_w
        pltpu.sync_copy(idx_hbm.at[pl.ds(base, b_per_w)], idx_v)

        def g_copy(c):
            return pltpu.make_async_copy(
                table_hbm.at[idx_v.at[pl.ds(c * CH, CH)]],
                rows_v.at[c % R], g_sems[c % R])

        def o_copy(c):
            return pltpu.make_async_copy(
                rows_v.at[c % R],
                out_hbm.at[pl.ds(base + c * CH, CH)], o_sems[c % R])

        for c in range(min(R, n_chunks)):
            g_copy(c).start()
        for c in range(n_chunks):
            g_copy(c).wait()
            o_copy(c).start()
            if c + R < n_chunks:
                o_copy(c).wait()
                g_copy(c + R).start()
        for c in range(max(n_chunks - R, 0), n_chunks):
            o_copy(c).wait()

    return gather_k


def _router_body(x_ref, wsw_ref, bsw_ref, routes_ref, rank_ref, counts_ref,
                 run_cnt):
    i = pl.program_id(0)

    @pl.when(i == 0)
    def _init():
        run_cnt[...] = jnp.zeros((1, N_EXPERTS), jnp.int32)

    x = x_ref[...]
    w = wsw_ref[...]
    logits = jax.lax.dot_general(
        x, w, (((1,), (1,)), ((), ())),
        preferred_element_type=jnp.float32) + bsw_ref[...]
    m = jnp.max(logits, axis=1, keepdims=True)
    iota = jax.lax.broadcasted_iota(jnp.int32, logits.shape, 1)
    cand = jnp.where(logits == m, iota, N_EXPERTS)
    routes = jnp.min(cand, axis=1, keepdims=True)
    routes_ref[...] = routes

    onehot = (routes == iota).astype(jnp.float32)
    r_iota = jax.lax.broadcasted_iota(jnp.int32, (BT, BT), 0)
    c_iota = jax.lax.broadcasted_iota(jnp.int32, (BT, BT), 1)
    tril = (r_iota >= c_iota).astype(jnp.float32)
    csum = jax.lax.dot_general(
        tril, onehot, (((1,), (0,)), ((), ())),
        preferred_element_type=jnp.float32)
    base = run_cnt[...].astype(jnp.float32)
    rank_f = jnp.sum((onehot * (csum - 1.0 + base)), axis=1, keepdims=True)
    rank_ref[...] = rank_f.astype(jnp.int32)
    new_cnt = (base + csum[-1:, :]).astype(jnp.int32)
    run_cnt[...] = new_cnt
    counts_ref[...] = new_cnt


def _ffn_body(eid_ref, xs_ref, w1_ref, b1_ref, w2_ref, b2_ref, wsw_ref,
              bsw_ref, out_ref):
    del eid_ref
    x = xs_ref[...]
    logits = jax.lax.dot_general(
        x, wsw_ref[...], (((1,), (1,)), ((), ())),
        preferred_element_type=jnp.float32) + bsw_ref[...]
    m = jnp.max(logits, axis=1, keepdims=True)
    scale = 1.0 / jnp.sum(jnp.exp(logits - m), axis=1, keepdims=True)

    h = jax.lax.dot_general(
        x, w1_ref[0], (((1,), (1,)), ((), ())),
        preferred_element_type=jnp.float32) + b1_ref[0]
    h = jnp.maximum(h, 0.0)
    o = jax.lax.dot_general(
        h, w2_ref[0], (((1,), (1,)), ((), ())),
        preferred_element_type=jnp.float32) + b2_ref[0]
    out_ref[...] = o * scale


def kernel(x, W_sw, b_sw, W1, b1, W2, b2):
    B, N, T, C = x.shape
    xf = x.reshape(-1, C)
    bsw2 = b_sw.reshape(1, N_EXPERTS)

    routes2, rank2, counts2 = pl.pallas_call(
        _router_body,
        grid=(NTOK // BT,),
        in_specs=[
            pl.BlockSpec((BT, C), lambda i: (i, 0)),
            pl.BlockSpec((N_EXPERTS, C), lambda i: (0, 0)),
            pl.BlockSpec((1, N_EXPERTS), lambda i: (0, 0)),
        ],
        out_specs=[
            pl.BlockSpec((BT, 1), lambda i: (i, 0)),
            pl.BlockSpec((BT, 1), lambda i: (i, 0)),
            pl.BlockSpec((1, N_EXPERTS), lambda i: (0, 0)),
        ],
        out_shape=[
            jax.ShapeDtypeStruct((NTOK, 1), jnp.int32),
            jax.ShapeDtypeStruct((NTOK, 1), jnp.int32),
            jax.ShapeDtypeStruct((1, N_EXPERTS), jnp.int32),
        ],
        scratch_shapes=[pltpu.VMEM((1, N_EXPERTS), jnp.int32)],
    )(xf, W_sw, bsw2)
    routes = routes2[:, 0]
    rank = rank2[:, 0]

    counts = counts2[0]
    blocks_e = (counts + BT - 1) // BT
    cum_blocks = jnp.cumsum(blocks_e)
    pad_off = cum_blocks - blocks_e
    k_blocks = cum_blocks[-1]

    bid = jnp.arange(NBLK, dtype=jnp.int32)
    eid_raw = jnp.sum((cum_blocks[None, :] <= bid[:, None]).astype(jnp.int32),
                      axis=1)
    last_eid = jnp.sum((cum_blocks <= (k_blocks - 1)).astype(jnp.int32))
    eids = jnp.where(bid < k_blocks, jnp.minimum(eid_raw, N_EXPERTS - 1),
                     last_eid).astype(jnp.int32)

    pad_off_tok = jnp.zeros((NTOK,), jnp.int32)
    for e in range(N_EXPERTS):
        pad_off_tok = jnp.where(routes == e, pad_off[e], pad_off_tok)
    pos = (pad_off_tok * BT + rank).astype(jnp.int32)
    src = jnp.zeros((NPAD,), jnp.int32).at[pos].set(
        jnp.arange(NTOK, dtype=jnp.int32))

    xs = _make_sc_row_gather(NTOK, C, NPAD)(xf, src)

    out_p = pl.pallas_call(
        _ffn_body,
        grid_spec=pltpu.PrefetchScalarGridSpec(
            num_scalar_prefetch=1,
            grid=(NBLK,),
            in_specs=[
                pl.BlockSpec((BT, C), lambda i, eid: (i, 0)),
                pl.BlockSpec((1, HIDDEN, C), lambda i, eid: (eid[i], 0, 0)),
                pl.BlockSpec((1, 1, HIDDEN), lambda i, eid: (eid[i], 0, 0)),
                pl.BlockSpec((1, C, HIDDEN), lambda i, eid: (eid[i], 0, 0)),
                pl.BlockSpec((1, 1, C), lambda i, eid: (eid[i], 0, 0)),
                pl.BlockSpec((N_EXPERTS, C), lambda i, eid: (0, 0)),
                pl.BlockSpec((1, N_EXPERTS), lambda i, eid: (0, 0)),
            ],
            out_specs=pl.BlockSpec((BT, C), lambda i, eid: (i, 0)),
        ),
        out_shape=jax.ShapeDtypeStruct((NBLK * BT, C), jnp.float32),
    )(eids, xs, W1, b1.reshape(N_EXPERTS, 1, HIDDEN), W2,
      b2.reshape(N_EXPERTS, 1, C), W_sw, bsw2)

    out = _make_sc_row_gather(NBLK * BT, C, NTOK)(out_p, pos)
    return out.reshape(B, N, T, C)


# speedup vs baseline: 1.5396x; 1.5396x over previous
import functools

import jax
import jax.numpy as jnp
from jax import lax
from jax.experimental import pallas as pl
from jax.experimental.pallas import tpu as pltpu
from jax.experimental.pallas import tpu_sc as plsc

IN_DIM = 1024
HIDDEN = 2048
N_EXPERTS = 8
BT = 256
NBLK = (4096 // BT) + N_EXPERTS - 1
NTOK = 4096
NPAD = (NBLK + 1) * BT


def _make_sc_row_gather(V, D, B):
    info = plsc.get_sparse_core_info()
    NC, NS = info.num_cores, info.num_subcores
    NW = NC * NS
    CH = 32
    R = 3
    assert B % (CH * NW) == 0
    b_per_w = B // NW
    n_chunks = b_per_w // CH
    mesh = plsc.VectorSubcoreMesh(core_axis_name="c", subcore_axis_name="s")

    @functools.partial(
        pl.kernel, mesh=mesh,
        out_type=jax.ShapeDtypeStruct((B, D), jnp.float32),
        scratch_types=[
            pltpu.VMEM((b_per_w,), jnp.int32),
            pltpu.VMEM((R, CH, D), jnp.float32),
        ] + [pltpu.SemaphoreType.DMA] * (2 * R),
    )
    def gather_k(table_hbm, idx_hbm, out_hbm, idx_v, rows_v, *sems):
        g_sems, o_sems = sems[:R], sems[R:]
        wid = lax.axis_index("s") * NC + lax.axis_index("c")
        base = wid * b_per_w
        pltpu.sync_copy(idx_hbm.at[pl.ds(base, b_per_w)], idx_v)

        def g_copy(c):
            return pltpu.make_async_copy(
                table_hbm.at[idx_v.at[pl.ds(c * CH, CH)]],
                rows_v.at[c % R], g_sems[c % R])

        def o_copy(c):
            return pltpu.make_async_copy(
                rows_v.at[c % R],
                out_hbm.at[pl.ds(base + c * CH, CH)], o_sems[c % R])

        for c in range(min(R, n_chunks)):
            g_copy(c).start()
        for c in range(n_chunks):
            g_copy(c).wait()
            o_copy(c).start()
            if c + R < n_chunks:
                o_copy(c).wait()
                g_copy(c + R).start()
        for c in range(max(n_chunks - R, 0), n_chunks):
            o_copy(c).wait()

    return gather_k


def _router_body(x_ref, wsw_ref, bsw_ref, routes_ref, rank_ref, counts_ref,
                 run_cnt):
    i = pl.program_id(0)

    @pl.when(i == 0)
    def _init():
        run_cnt[...] = jnp.zeros((1, N_EXPERTS), jnp.int32)

    x = x_ref[...]
    w = wsw_ref[...]
    logits = jax.lax.dot_general(
        x, w, (((1,), (1,)), ((), ())),
        preferred_element_type=jnp.float32) + bsw_ref[...]
    m = jnp.max(logits, axis=1, keepdims=True)
    iota = jax.lax.broadcasted_iota(jnp.int32, logits.shape, 1)
    cand = jnp.where(logits == m, iota, N_EXPERTS)
    routes = jnp.min(cand, axis=1, keepdims=True)
    routes_ref[...] = routes

    onehot = (routes == iota).astype(jnp.float32)
    r_iota = jax.lax.broadcasted_iota(jnp.int32, (BT, BT), 0)
    c_iota = jax.lax.broadcasted_iota(jnp.int32, (BT, BT), 1)
    tril = (r_iota >= c_iota).astype(jnp.float32)
    csum = jax.lax.dot_general(
        tril, onehot, (((1,), (0,)), ((), ())),
        preferred_element_type=jnp.float32)
    base = run_cnt[...].astype(jnp.float32)
    rank_f = jnp.sum((onehot * (csum - 1.0 + base)), axis=1, keepdims=True)
    rank_ref[...] = rank_f.astype(jnp.int32)
    new_cnt = (base + csum[-1:, :]).astype(jnp.int32)
    run_cnt[...] = new_cnt
    counts_ref[...] = new_cnt


def _ffn_body(eid_ref, xs_ref, w1_ref, b1_ref, w2_ref, b2_ref, wsw_ref,
              bsw_ref, out_ref):
    del eid_ref
    x = xs_ref[...]
    logits = jax.lax.dot_general(
        x, wsw_ref[...], (((1,), (1,)), ((), ())),
        preferred_element_type=jnp.float32) + bsw_ref[...]
    m = jnp.max(logits, axis=1, keepdims=True)
    scale = 1.0 / jnp.sum(jnp.exp(logits - m), axis=1, keepdims=True)

    h = jax.lax.dot_general(
        x, w1_ref[0], (((1,), (1,)), ((), ())),
        preferred_element_type=jnp.float32) + b1_ref[0]
    h = jnp.maximum(h, 0.0)
    o = jax.lax.dot_general(
        h, w2_ref[0], (((1,), (1,)), ((), ())),
        preferred_element_type=jnp.float32) + b2_ref[0]
    out_ref[...] = o * scale


def kernel(x, W_sw, b_sw, W1, b1, W2, b2):
    B, N, T, C = x.shape
    xf = x.reshape(-1, C)
    bsw2 = b_sw.reshape(1, N_EXPERTS)

    routes2, rank2, counts2 = pl.pallas_call(
        _router_body,
        grid=(NTOK // BT,),
        in_specs=[
            pl.BlockSpec((BT, C), lambda i: (i, 0)),
            pl.BlockSpec((N_EXPERTS, C), lambda i: (0, 0)),
            pl.BlockSpec((1, N_EXPERTS), lambda i: (0, 0)),
        ],
        out_specs=[
            pl.BlockSpec((BT, 1), lambda i: (i, 0)),
            pl.BlockSpec((BT, 1), lambda i: (i, 0)),
            pl.BlockSpec((1, N_EXPERTS), lambda i: (0, 0)),
        ],
        out_shape=[
            jax.ShapeDtypeStruct((NTOK, 1), jnp.int32),
            jax.ShapeDtypeStruct((NTOK, 1), jnp.int32),
            jax.ShapeDtypeStruct((1, N_EXPERTS), jnp.int32),
        ],
        scratch_shapes=[pltpu.VMEM((1, N_EXPERTS), jnp.int32)],
    )(xf, W_sw, bsw2)
    routes = routes2[:, 0]
    rank = rank2[:, 0]

    counts = counts2[0]
    blocks_e = (counts + BT - 1) // BT
    cum_blocks = jnp.cumsum(blocks_e)
    pad_off = cum_blocks - blocks_e
    k_blocks = cum_blocks[-1]

    bid = jnp.arange(NBLK, dtype=jnp.int32)
    eid_raw = jnp.sum((cum_blocks[None, :] <= bid[:, None]).astype(jnp.int32),
                      axis=1)
    last_eid = jnp.sum((cum_blocks <= (k_blocks - 1)).astype(jnp.int32))
    eids = jnp.where(bid < k_blocks, jnp.minimum(eid_raw, N_EXPERTS - 1),
                     last_eid).astype(jnp.int32)

    pad_off_tok = jnp.zeros((NTOK,), jnp.int32)
    for e in range(N_EXPERTS):
        pad_off_tok = jnp.where(routes == e, pad_off[e], pad_off_tok)
    pos = (pad_off_tok * BT + rank).astype(jnp.int32)
    pad_fill = jnp.arange(NPAD, dtype=jnp.int32) % NTOK
    src = pad_fill.at[pos].set(jnp.arange(NTOK, dtype=jnp.int32))

    xs = _make_sc_row_gather(NTOK, C, NPAD)(xf, src)

    out_p = pl.pallas_call(
        _ffn_body,
        grid_spec=pltpu.PrefetchScalarGridSpec(
            num_scalar_prefetch=1,
            grid=(NBLK,),
            in_specs=[
                pl.BlockSpec((BT, C), lambda i, eid: (i, 0)),
                pl.BlockSpec((1, HIDDEN, C), lambda i, eid: (eid[i], 0, 0)),
                pl.BlockSpec((1, 1, HIDDEN), lambda i, eid: (eid[i], 0, 0)),
                pl.BlockSpec((1, C, HIDDEN), lambda i, eid: (eid[i], 0, 0)),
                pl.BlockSpec((1, 1, C), lambda i, eid: (eid[i], 0, 0)),
                pl.BlockSpec((N_EXPERTS, C), lambda i, eid: (0, 0)),
                pl.BlockSpec((1, N_EXPERTS), lambda i, eid: (0, 0)),
            ],
            out_specs=pl.BlockSpec((BT, C), lambda i, eid: (i, 0)),
        ),
        out_shape=jax.ShapeDtypeStruct((NBLK * BT, C), jnp.float32),
    )(eids, xs, W1, b1.reshape(N_EXPERTS, 1, HIDDEN), W2,
      b2.reshape(N_EXPERTS, 1, C), W_sw, bsw2)

    out = _make_sc_row_gather(NBLK * BT, C, NTOK)(out_p, pos)
    return out.reshape(B, N, T, C)


# speedup vs baseline: 1.7325x; 1.1253x over previous
import functools

import jax
import jax.numpy as jnp
from jax import lax
from jax.experimental import pallas as pl
from jax.experimental.pallas import tpu as pltpu
from jax.experimental.pallas import tpu_sc as plsc

IN_DIM = 1024
HIDDEN = 2048
N_EXPERTS = 8
BT = 256
NBLK = (4096 // BT) + N_EXPERTS - 1
NTOK = 4096


def _make_sc_row_gather(V, D, B):
    info = plsc.get_sparse_core_info()
    NC, NS = info.num_cores, info.num_subcores
    NW = NC * NS
    CH = 32
    R = 3
    assert B % (CH * NW) == 0
    b_per_w = B // NW
    n_chunks = b_per_w // CH
    mesh = plsc.VectorSubcoreMesh(core_axis_name="c", subcore_axis_name="s")

    @functools.partial(
        pl.kernel, mesh=mesh,
        out_type=jax.ShapeDtypeStruct((B, D), jnp.float32),
        scratch_types=[
            pltpu.VMEM((b_per_w,), jnp.int32),
            pltpu.VMEM((R, CH, D), jnp.float32),
        ] + [pltpu.SemaphoreType.DMA] * (2 * R),
    )
    def gather_k(table_hbm, idx_hbm, out_hbm, idx_v, rows_v, *sems):
        g_sems, o_sems = sems[:R], sems[R:]
        wid = lax.axis_index("s") * NC + lax.axis_index("c")
        base = wid * b_per_w
        pltpu.sync_copy(idx_hbm.at[pl.ds(base, b_per_w)], idx_v)

        def g_copy(c):
            return pltpu.make_async_copy(
                table_hbm.at[idx_v.at[pl.ds(c * CH, CH)]],
                rows_v.at[c % R], g_sems[c % R])

        def o_copy(c):
            return pltpu.make_async_copy(
                rows_v.at[c % R],
                out_hbm.at[pl.ds(base + c * CH, CH)], o_sems[c % R])

        for c in range(min(R, n_chunks)):
            g_copy(c).start()
        for c in range(n_chunks):
            g_copy(c).wait()
            o_copy(c).start()
            if c + R < n_chunks:
                o_copy(c).wait()
                g_copy(c + R).start()
        for c in range(max(n_chunks - R, 0), n_chunks):
            o_copy(c).wait()

    return gather_k


def _make_sc_row_scatter(B, D, V):
    info = plsc.get_sparse_core_info()
    NC, NS = info.num_cores, info.num_subcores
    NW = NC * NS
    CH = 32
    R = 3
    assert B % (CH * NW) == 0
    b_per_w = B // NW
    n_chunks = b_per_w // CH
    mesh = plsc.VectorSubcoreMesh(core_axis_name="c", subcore_axis_name="s")

    @functools.partial(
        pl.kernel, mesh=mesh,
        out_type=jax.ShapeDtypeStruct((V, D), jnp.float32),
        scratch_types=[
            pltpu.VMEM((n_chunks, CH), jnp.int32),
            pltpu.VMEM((R, CH, D), jnp.float32),
        ] + [pltpu.SemaphoreType.DMA] * (2 * R),
    )
    def scatter_k(data_hbm, idx_hbm, out_hbm, idx_v, rows_v, *sems):
        g_sems, s_sems = sems[:R], sems[R:]
        wid = lax.axis_index("s") * NC + lax.axis_index("c")
        base = wid * b_per_w
        for c in range(n_chunks):
            pltpu.sync_copy(idx_hbm.at[pl.ds(base + c * CH, CH)],
                            idx_v.at[c])

        def g_copy(c):
            return pltpu.make_async_copy(
                data_hbm.at[pl.ds(base + c * CH, CH)],
                rows_v.at[c % R], g_sems[c % R])

        def s_copy(c):
            return pltpu.make_async_copy(
                rows_v.at[c % R], out_hbm.at[idx_v.at[c]], s_sems[c % R])

        for c in range(min(R, n_chunks)):
            g_copy(c).start()
        for c in range(n_chunks):
            g_copy(c).wait()
            s_copy(c).start()
            if c + R < n_chunks:
                s_copy(c).wait()
                g_copy(c + R).start()
        for c in range(max(n_chunks - R, 0), n_chunks):
            s_copy(c).wait()

    return scatter_k


def _router_body(x_ref, wsw_ref, bsw_ref, routes_ref, rank_ref, counts_ref,
                 run_cnt):
    i = pl.program_id(0)

    @pl.when(i == 0)
    def _init():
        run_cnt[...] = jnp.zeros((1, N_EXPERTS), jnp.int32)

    x = x_ref[...]
    w = wsw_ref[...]
    logits = jax.lax.dot_general(
        x, w, (((1,), (1,)), ((), ())),
        preferred_element_type=jnp.float32) + bsw_ref[...]
    m = jnp.max(logits, axis=1, keepdims=True)
    iota = jax.lax.broadcasted_iota(jnp.int32, logits.shape, 1)
    cand = jnp.where(logits == m, iota, N_EXPERTS)
    routes = jnp.min(cand, axis=1, keepdims=True)
    routes_ref[...] = routes

    onehot = (routes == iota).astype(jnp.float32)
    r_iota = jax.lax.broadcasted_iota(jnp.int32, (BT, BT), 0)
    c_iota = jax.lax.broadcasted_iota(jnp.int32, (BT, BT), 1)
    tril = (r_iota >= c_iota).astype(jnp.float32)
    csum = jax.lax.dot_general(
        tril, onehot, (((1,), (0,)), ((), ())),
        preferred_element_type=jnp.float32)
    base = run_cnt[...].astype(jnp.float32)
    rank_f = jnp.sum((onehot * (csum - 1.0 + base)), axis=1, keepdims=True)
    rank_ref[...] = rank_f.astype(jnp.int32)
    new_cnt = (base + csum[-1:, :]).astype(jnp.int32)
    run_cnt[...] = new_cnt
    counts_ref[...] = new_cnt


def _ffn_body(eid_ref, xs_ref, w1_ref, b1_ref, w2_ref, b2_ref, wsw_ref,
              bsw_ref, out_ref):
    del eid_ref
    x = xs_ref[...]
    logits = jax.lax.dot_general(
        x, wsw_ref[...], (((1,), (1,)), ((), ())),
        preferred_element_type=jnp.float32) + bsw_ref[...]
    m = jnp.max(logits, axis=1, keepdims=True)
    scale = 1.0 / jnp.sum(jnp.exp(logits - m), axis=1, keepdims=True)

    h = jax.lax.dot_general(
        x, w1_ref[0], (((1,), (1,)), ((), ())),
        preferred_element_type=jnp.float32) + b1_ref[0]
    h = jnp.maximum(h, 0.0)
    o = jax.lax.dot_general(
        h, w2_ref[0], (((1,), (1,)), ((), ())),
        preferred_element_type=jnp.float32) + b2_ref[0]
    out_ref[...] = o * scale


def kernel(x, W_sw, b_sw, W1, b1, W2, b2):
    B, N, T, C = x.shape
    xf = x.reshape(-1, C)
    bsw2 = b_sw.reshape(1, N_EXPERTS)

    routes2, rank2, counts2 = pl.pallas_call(
        _router_body,
        grid=(NTOK // BT,),
        in_specs=[
            pl.BlockSpec((BT, C), lambda i: (i, 0)),
            pl.BlockSpec((N_EXPERTS, C), lambda i: (0, 0)),
            pl.BlockSpec((1, N_EXPERTS), lambda i: (0, 0)),
        ],
        out_specs=[
            pl.BlockSpec((BT, 1), lambda i: (i, 0)),
            pl.BlockSpec((BT, 1), lambda i: (i, 0)),
            pl.BlockSpec((1, N_EXPERTS), lambda i: (0, 0)),
        ],
        out_shape=[
            jax.ShapeDtypeStruct((NTOK, 1), jnp.int32),
            jax.ShapeDtypeStruct((NTOK, 1), jnp.int32),
            jax.ShapeDtypeStruct((1, N_EXPERTS), jnp.int32),
        ],
        scratch_shapes=[pltpu.VMEM((1, N_EXPERTS), jnp.int32)],
    )(xf, W_sw, bsw2)
    routes = routes2[:, 0]
    rank = rank2[:, 0]

    counts = counts2[0]
    blocks_e = (counts + BT - 1) // BT
    cum_blocks = jnp.cumsum(blocks_e)
    pad_off = cum_blocks - blocks_e
    k_blocks = cum_blocks[-1]

    bid = jnp.arange(NBLK, dtype=jnp.int32)
    eid_raw = jnp.sum((cum_blocks[None, :] <= bid[:, None]).astype(jnp.int32),
                      axis=1)
    last_eid = jnp.sum((cum_blocks <= (k_blocks - 1)).astype(jnp.int32))
    eids = jnp.where(bid < k_blocks, jnp.minimum(eid_raw, N_EXPERTS - 1),
                     last_eid).astype(jnp.int32)

    pad_off_tok = jnp.zeros((NTOK,), jnp.int32)
    for e in range(N_EXPERTS):
        pad_off_tok = jnp.where(routes == e, pad_off[e], pad_off_tok)
    pos = (pad_off_tok * BT + rank).astype(jnp.int32)

    xs = _make_sc_row_scatter(NTOK, C, NBLK * BT)(xf, pos)

    out_p = pl.pallas_call(
        _ffn_body,
        grid_spec=pltpu.PrefetchScalarGridSpec(
            num_scalar_prefetch=1,
            grid=(NBLK,),
            in_specs=[
                pl.BlockSpec((BT, C), lambda i, eid: (i, 0)),
                pl.BlockSpec((1, HIDDEN, C), lambda i, eid: (eid[i], 0, 0)),
                pl.BlockSpec((1, 1, HIDDEN), lambda i, eid: (eid[i], 0, 0)),
                pl.BlockSpec((1, C, HIDDEN), lambda i, eid: (eid[i], 0, 0)),
                pl.BlockSpec((1, 1, C), lambda i, eid: (eid[i], 0, 0)),
                pl.BlockSpec((N_EXPERTS, C), lambda i, eid: (0, 0)),
                pl.BlockSpec((1, N_EXPERTS), lambda i, eid: (0, 0)),
            ],
            out_specs=pl.BlockSpec((BT, C), lambda i, eid: (i, 0)),
        ),
        out_shape=jax.ShapeDtypeStruct((NBLK * BT, C), jnp.float32),
    )(eids, xs, W1, b1.reshape(N_EXPERTS, 1, HIDDEN), W2,
      b2.reshape(N_EXPERTS, 1, C), W_sw, bsw2)

    out = _make_sc_row_gather(NBLK * BT, C, NTOK)(out_p, pos)
    return out.reshape(B, N, T, C)


# speedup vs baseline: 1.8879x; 1.0897x over previous
import functools

import jax
import jax.numpy as jnp
from jax import lax
from jax.experimental import pallas as pl
from jax.experimental.pallas import tpu as pltpu
from jax.experimental.pallas import tpu_sc as plsc

IN_DIM = 1024
HIDDEN = 2048
N_EXPERTS = 8
BT = 256
NBLK = (4096 // BT) + N_EXPERTS - 1
NTOK = 4096


def _make_sc_row_gather(V, D, B):
    info = plsc.get_sparse_core_info()
    NC, NS = info.num_cores, info.num_subcores
    NW = NC * NS
    CH = 32
    R = 3
    assert B % (CH * NW) == 0
    b_per_w = B // NW
    n_chunks = b_per_w // CH
    mesh = plsc.VectorSubcoreMesh(core_axis_name="c", subcore_axis_name="s")

    @functools.partial(
        pl.kernel, mesh=mesh,
        out_type=jax.ShapeDtypeStruct((B, D), jnp.float32),
        scratch_types=[
            pltpu.VMEM((b_per_w,), jnp.int32),
            pltpu.VMEM((R, CH, D), jnp.float32),
        ] + [pltpu.SemaphoreType.DMA] * (2 * R),
    )
    def gather_k(table_hbm, idx_hbm, out_hbm, idx_v, rows_v, *sems):
        g_sems, o_sems = sems[:R], sems[R:]
        wid = lax.axis_index("s") * NC + lax.axis_index("c")
        base = wid * b_per_w
        pltpu.sync_copy(idx_hbm.at[pl.ds(base, b_per_w)], idx_v)

        def g_copy(c):
            return pltpu.make_async_copy(
                table_hbm.at[idx_v.at[pl.ds(c * CH, CH)]],
                rows_v.at[c % R], g_sems[c % R])

        def o_copy(c):
            return pltpu.make_async_copy(
                rows_v.at[c % R],
                out_hbm.at[pl.ds(base + c * CH, CH)], o_sems[c % R])

        for c in range(min(R, n_chunks)):
            g_copy(c).start()
        for c in range(n_chunks):
            g_copy(c).wait()
            o_copy(c).start()
            if c + R < n_chunks:
                o_copy(c).wait()
                g_copy(c + R).start()
        for c in range(max(n_chunks - R, 0), n_chunks):
            o_copy(c).wait()

    return gather_k


def _make_sc_row_scatter(B, D, V):
    info = plsc.get_sparse_core_info()
    NC, NS = info.num_cores, info.num_subcores
    NW = NC * NS
    CH = 32
    R = 3
    assert B % (CH * NW) == 0
    b_per_w = B // NW
    n_chunks = b_per_w // CH
    mesh = plsc.VectorSubcoreMesh(core_axis_name="c", subcore_axis_name="s")

    @functools.partial(
        pl.kernel, mesh=mesh,
        out_type=jax.ShapeDtypeStruct((V, D), jnp.float32),
        scratch_types=[
            pltpu.VMEM((n_chunks, CH), jnp.int32),
            pltpu.VMEM((R, CH, D), jnp.float32),
        ] + [pltpu.SemaphoreType.DMA] * (2 * R),
    )
    def scatter_k(data_hbm, idx_hbm, out_hbm, idx_v, rows_v, *sems):
        g_sems, s_sems = sems[:R], sems[R:]
        wid = lax.axis_index("s") * NC + lax.axis_index("c")
        base = wid * b_per_w
        for c in range(n_chunks):
            pltpu.sync_copy(idx_hbm.at[pl.ds(base + c * CH, CH)],
                            idx_v.at[c])

        def g_copy(c):
            return pltpu.make_async_copy(
                data_hbm.at[pl.ds(base + c * CH, CH)],
                rows_v.at[c % R], g_sems[c % R])

        def s_copy(c):
            return pltpu.make_async_copy(
                rows_v.at[c % R], out_hbm.at[idx_v.at[c]], s_sems[c % R])

        for c in range(min(R, n_chunks)):
            g_copy(c).start()
        for c in range(n_chunks):
            g_copy(c).wait()
            s_copy(c).start()
            if c + R < n_chunks:
                s_copy(c).wait()
                g_copy(c + R).start()
        for c in range(max(n_chunks - R, 0), n_chunks):
            s_copy(c).wait()

    return scatter_k


def _router_body(x_ref, wsw_ref, bsw_ref, pos_ref, counts_ref,
                 run_cnt, routes_s, rank_s):
    i = pl.program_id(0)

    @pl.when(i == 0)
    def _init():
        run_cnt[...] = jnp.zeros((1, N_EXPERTS), jnp.int32)

    @pl.when(i < NTOK // BT)
    def _route():
        x = x_ref[...]
        w = wsw_ref[...]
        logits = jax.lax.dot_general(
            x, w, (((1,), (1,)), ((), ())),
            preferred_element_type=jnp.float32) + bsw_ref[...]
        m = jnp.max(logits, axis=1, keepdims=True)
        iota = jax.lax.broadcasted_iota(jnp.int32, logits.shape, 1)
        cand = jnp.where(logits == m, iota, N_EXPERTS)
        routes = jnp.min(cand, axis=1, keepdims=True)

        onehot = (routes == iota).astype(jnp.float32)
        r_iota = jax.lax.broadcasted_iota(jnp.int32, (BT, BT), 0)
        c_iota = jax.lax.broadcasted_iota(jnp.int32, (BT, BT), 1)
        tril = (r_iota >= c_iota).astype(jnp.float32)
        csum = jax.lax.dot_general(
            tril, onehot, (((1,), (0,)), ((), ())),
            preferred_element_type=jnp.float32)
        base = run_cnt[...].astype(jnp.float32)
        rank_f = jnp.sum(onehot * (csum - 1.0 + base), axis=1,
                         keepdims=True)
        routes_s[pl.ds(i * BT, BT), :] = routes
        rank_s[pl.ds(i * BT, BT), :] = rank_f.astype(jnp.int32)
        new_cnt = (base + csum[-1:, :]).astype(jnp.int32)
        run_cnt[...] = new_cnt
        counts_ref[...] = new_cnt

    @pl.when(i == NTOK // BT)
    def _finalize():
        counts = run_cnt[...].astype(jnp.float32)
        blocks = jnp.floor((counts + (BT - 1)) * (1.0 / BT))
        e_r = jax.lax.broadcasted_iota(jnp.int32, (N_EXPERTS, N_EXPERTS), 0)
        e_c = jax.lax.broadcasted_iota(jnp.int32, (N_EXPERTS, N_EXPERTS), 1)
        tri = (e_r <= e_c).astype(jnp.float32)
        cum_blocks = jax.lax.dot_general(
            blocks, tri, (((1,), (0,)), ((), ())),
            preferred_element_type=jnp.float32)
        pad_off = cum_blocks - blocks
        routes_all = routes_s[...]
        iota_all = jax.lax.broadcasted_iota(
            jnp.int32, (NTOK, N_EXPERTS), 1)
        onehot_all = (routes_all == iota_all).astype(jnp.float32)
        pad_tok = jax.lax.dot_general(
            onehot_all, pad_off, (((1,), (1,)), ((), ())),
            preferred_element_type=jnp.float32)
        pos_f = pad_tok * float(BT) + rank_s[...].astype(jnp.float32)
        pos_ref[...] = pos_f.astype(jnp.int32)


def _ffn_body(eid_ref, xs_ref, w1_ref, b1_ref, w2_ref, b2_ref, wsw_ref,
              bsw_ref, out_ref):
    i = pl.program_id(0)

    @pl.when(i < eid_ref[NBLK])
    def _compute():
        x = xs_ref[...]
        logits = jax.lax.dot_general(
            x, wsw_ref[...], (((1,), (1,)), ((), ())),
            preferred_element_type=jnp.float32) + bsw_ref[...]
        m = jnp.max(logits, axis=1, keepdims=True)
        scale = 1.0 / jnp.sum(jnp.exp(logits - m), axis=1, keepdims=True)

        h = jax.lax.dot_general(
            x, w1_ref[0], (((1,), (1,)), ((), ())),
            preferred_element_type=jnp.float32) + b1_ref[0]
        h = jnp.maximum(h, 0.0)
        o = jax.lax.dot_general(
            h, w2_ref[0], (((1,), (1,)), ((), ())),
            preferred_element_type=jnp.float32) + b2_ref[0]
        out_ref[...] = o * scale


def kernel(x, W_sw, b_sw, W1, b1, W2, b2):
    B, N, T, C = x.shape
    xf = x.reshape(-1, C)
    bsw2 = b_sw.reshape(1, N_EXPERTS)

    pos2, counts2 = pl.pallas_call(
        _router_body,
        grid=(NTOK // BT + 1,),
        in_specs=[
            pl.BlockSpec((BT, C), lambda i: (jnp.minimum(i, NTOK // BT - 1),
                                             0)),
            pl.BlockSpec((N_EXPERTS, C), lambda i: (0, 0)),
            pl.BlockSpec((1, N_EXPERTS), lambda i: (0, 0)),
        ],
        out_specs=[
            pl.BlockSpec((NTOK, 1), lambda i: (0, 0)),
            pl.BlockSpec((1, N_EXPERTS), lambda i: (0, 0)),
        ],
        out_shape=[
            jax.ShapeDtypeStruct((NTOK, 1), jnp.int32),
            jax.ShapeDtypeStruct((1, N_EXPERTS), jnp.int32),
        ],
        scratch_shapes=[
            pltpu.VMEM((1, N_EXPERTS), jnp.int32),
            pltpu.VMEM((NTOK, 1), jnp.int32),
            pltpu.VMEM((NTOK, 1), jnp.int32),
        ],
    )(xf, W_sw, bsw2)
    pos = pos2[:, 0]

    counts = counts2[0]
    blocks_e = (counts + BT - 1) // BT
    cum_blocks = jnp.cumsum(blocks_e)
    pad_off = cum_blocks - blocks_e
    k_blocks = cum_blocks[-1]

    bid = jnp.arange(NBLK, dtype=jnp.int32)
    eid_raw = jnp.sum((cum_blocks[None, :] <= bid[:, None]).astype(jnp.int32),
                      axis=1)
    last_eid = jnp.sum((cum_blocks <= (k_blocks - 1)).astype(jnp.int32))
    eids = jnp.where(bid < k_blocks, jnp.minimum(eid_raw, N_EXPERTS - 1),
                     last_eid).astype(jnp.int32)
    eids_ext = jnp.concatenate([eids, k_blocks[None].astype(jnp.int32)])

    xs = _make_sc_row_scatter(NTOK, C, NBLK * BT)(xf, pos)

    out_p = pl.pallas_call(
        _ffn_body,
        grid_spec=pltpu.PrefetchScalarGridSpec(
            num_scalar_prefetch=1,
            grid=(NBLK,),
            in_specs=[
                pl.BlockSpec((BT, C), lambda i, eid: (i, 0)),
                pl.BlockSpec((1, HIDDEN, C), lambda i, eid: (eid[i], 0, 0)),
                pl.BlockSpec((1, 1, HIDDEN), lambda i, eid: (eid[i], 0, 0)),
                pl.BlockSpec((1, C, HIDDEN), lambda i, eid: (eid[i], 0, 0)),
                pl.BlockSpec((1, 1, C), lambda i, eid: (eid[i], 0, 0)),
                pl.BlockSpec((N_EXPERTS, C), lambda i, eid: (0, 0)),
                pl.BlockSpec((1, N_EXPERTS), lambda i, eid: (0, 0)),
            ],
            out_specs=pl.BlockSpec((BT, C), lambda i, eid: (i, 0)),
        ),
        out_shape=jax.ShapeDtypeStruct((NBLK * BT, C), jnp.float32),
    )(eids_ext, xs, W1, b1.reshape(N_EXPERTS, 1, HIDDEN), W2,
      b2.reshape(N_EXPERTS, 1, C), W_sw, bsw2)

    out = _make_sc_row_gather(NBLK * BT, C, NTOK)(out_p, pos)
    return out.reshape(B, N, T, C)


# speedup vs baseline: 1.8886x; 1.0004x over previous
import functools

import jax
import jax.numpy as jnp
from jax import lax
from jax.experimental import pallas as pl
from jax.experimental.pallas import tpu as pltpu
from jax.experimental.pallas import tpu_sc as plsc

IN_DIM = 1024
HIDDEN = 2048
N_EXPERTS = 8
BT = 256
NBLK = (4096 // BT) + N_EXPERTS - 1
NTOK = 4096


def _make_sc_row_gather(V, D, B):
    info = plsc.get_sparse_core_info()
    NC, NS = info.num_cores, info.num_subcores
    NW = NC * NS
    CH = 32
    R = 3
    assert B % (CH * NW) == 0
    b_per_w = B // NW
    n_chunks = b_per_w // CH
    mesh = plsc.VectorSubcoreMesh(core_axis_name="c", subcore_axis_name="s")

    @functools.partial(
        pl.kernel, mesh=mesh,
        out_type=jax.ShapeDtypeStruct((B, D), jnp.float32),
        scratch_types=[
            pltpu.VMEM((b_per_w,), jnp.int32),
            pltpu.VMEM((R, CH, D), jnp.float32),
        ] + [pltpu.SemaphoreType.DMA] * (2 * R),
    )
    def gather_k(table_hbm, idx_hbm, out_hbm, idx_v, rows_v, *sems):
        g_sems, o_sems = sems[:R], sems[R:]
        wid = lax.axis_index("s") * NC + lax.axis_index("c")
        base = wid * b_per_w
        pltpu.sync_copy(idx_hbm.at[pl.ds(base, b_per_w)], idx_v)

        def g_copy(c):
            return pltpu.make_async_copy(
                table_hbm.at[idx_v.at[pl.ds(c * CH, CH)]],
                rows_v.at[c % R], g_sems[c % R])

        def o_copy(c):
            return pltpu.make_async_copy(
                rows_v.at[c % R],
                out_hbm.at[pl.ds(base + c * CH, CH)], o_sems[c % R])

        for c in range(min(R, n_chunks)):
            g_copy(c).start()
        for c in range(n_chunks):
            g_copy(c).wait()
            o_copy(c).start()
            if c + R < n_chunks:
                o_copy(c).wait()
                g_copy(c + R).start()
        for c in range(max(n_chunks - R, 0), n_chunks):
            o_copy(c).wait()

    return gather_k


def _make_sc_row_scatter(B, D, V):
    info = plsc.get_sparse_core_info()
    NC, NS = info.num_cores, info.num_subcores
    NW = NC * NS
    CH = 32
    R = 3
    assert B % (CH * NW) == 0
    b_per_w = B // NW
    n_chunks = b_per_w // CH
    mesh = plsc.VectorSubcoreMesh(core_axis_name="c", subcore_axis_name="s")

    @functools.partial(
        pl.kernel, mesh=mesh,
        out_type=jax.ShapeDtypeStruct((V, D), jnp.float32),
        scratch_types=[
            pltpu.VMEM((n_chunks, CH), jnp.int32),
            pltpu.VMEM((R, CH, D), jnp.float32),
        ] + [pltpu.SemaphoreType.DMA] * (2 * R),
    )
    def scatter_k(data_hbm, idx_hbm, out_hbm, idx_v, rows_v, *sems):
        g_sems, s_sems = sems[:R], sems[R:]
        wid = lax.axis_index("s") * NC + lax.axis_index("c")
        base = wid * b_per_w
        for c in range(n_chunks):
            pltpu.sync_copy(idx_hbm.at[pl.ds(base + c * CH, CH)],
                            idx_v.at[c])

        def g_copy(c):
            return pltpu.make_async_copy(
                data_hbm.at[pl.ds(base + c * CH, CH)],
                rows_v.at[c % R], g_sems[c % R])

        def s_copy(c):
            return pltpu.make_async_copy(
                rows_v.at[c % R], out_hbm.at[idx_v.at[c]], s_sems[c % R])

        for c in range(min(R, n_chunks)):
            g_copy(c).start()
        for c in range(n_chunks):
            g_copy(c).wait()
            s_copy(c).start()
            if c + R < n_chunks:
                s_copy(c).wait()
                g_copy(c + R).start()
        for c in range(max(n_chunks - R, 0), n_chunks):
            s_copy(c).wait()

    return scatter_k


def _router_body(x_ref, wsw_ref, bsw_ref, pos_ref, counts_ref,
                 run_cnt, routes_s, rank_s):
    i = pl.program_id(0)

    @pl.when(i == 0)
    def _init():
        run_cnt[...] = jnp.zeros((1, N_EXPERTS), jnp.int32)

    @pl.when(i < NTOK // BT)
    def _route():
        x = x_ref[...]
        w = wsw_ref[...]
        logits = jax.lax.dot_general(
            x, w, (((1,), (1,)), ((), ())),
            preferred_element_type=jnp.float32) + bsw_ref[...]
        m = jnp.max(logits, axis=1, keepdims=True)
        iota = jax.lax.broadcasted_iota(jnp.int32, logits.shape, 1)
        cand = jnp.where(logits == m, iota, N_EXPERTS)
        routes = jnp.min(cand, axis=1, keepdims=True)

        onehot = (routes == iota).astype(jnp.float32)
        r_iota = jax.lax.broadcasted_iota(jnp.int32, (BT, BT), 0)
        c_iota = jax.lax.broadcasted_iota(jnp.int32, (BT, BT), 1)
        tril = (r_iota >= c_iota).astype(jnp.float32)
        csum = jax.lax.dot_general(
            tril, onehot, (((1,), (0,)), ((), ())),
            preferred_element_type=jnp.float32)
        base = run_cnt[...].astype(jnp.float32)
        rank_f = jnp.sum(onehot * (csum - 1.0 + base), axis=1,
                         keepdims=True)
        routes_s[pl.ds(i * BT, BT), :] = routes
        rank_s[pl.ds(i * BT, BT), :] = rank_f.astype(jnp.int32)
        new_cnt = (base + csum[-1:, :]).astype(jnp.int32)
        run_cnt[...] = new_cnt
        counts_ref[...] = new_cnt

    @pl.when(i == NTOK // BT)
    def _finalize():
        counts = run_cnt[...].astype(jnp.float32)
        blocks = jnp.floor((counts + (BT - 1)) * (1.0 / BT))
        e_r = jax.lax.broadcasted_iota(jnp.int32, (N_EXPERTS, N_EXPERTS), 0)
        e_c = jax.lax.broadcasted_iota(jnp.int32, (N_EXPERTS, N_EXPERTS), 1)
        tri = (e_r <= e_c).astype(jnp.float32)
        cum_blocks = jax.lax.dot_general(
            blocks, tri, (((1,), (0,)), ((), ())),
            preferred_element_type=jnp.float32)
        pad_off = cum_blocks - blocks
        routes_all = routes_s[...]
        iota_all = jax.lax.broadcasted_iota(
            jnp.int32, (NTOK, N_EXPERTS), 1)
        onehot_all = (routes_all == iota_all).astype(jnp.float32)
        pad_tok = jax.lax.dot_general(
            onehot_all, pad_off, (((1,), (1,)), ((), ())),
            preferred_element_type=jnp.float32)
        pos_f = pad_tok * float(BT) + rank_s[...].astype(jnp.float32)
        pos_ref[...] = pos_f.astype(jnp.int32)


def _ffn_body(eid_ref, xs_ref, w1_ref, b1_ref, w2_ref, b2_ref, wsw_ref,
              bsw_ref, out_ref):
    i = pl.program_id(0)

    @pl.when(i < eid_ref[NBLK])
    def _compute():
        x = xs_ref[...]
        logits = jax.lax.dot_general(
            x, wsw_ref[...], (((1,), (1,)), ((), ())),
            preferred_element_type=jnp.float32) + bsw_ref[...]
        m = jnp.max(logits, axis=1, keepdims=True)
        scale = 1.0 / jnp.sum(jnp.exp(logits - m), axis=1, keepdims=True)

        xb = x.astype(jnp.bfloat16)
        h = jax.lax.dot_general(
            xb, w1_ref[0].astype(jnp.bfloat16), (((1,), (1,)), ((), ())),
            preferred_element_type=jnp.float32) + b1_ref[0]
        h = jnp.maximum(h, 0.0)
        o = jax.lax.dot_general(
            h.astype(jnp.bfloat16), w2_ref[0].astype(jnp.bfloat16),
            (((1,), (1,)), ((), ())),
            preferred_element_type=jnp.float32) + b2_ref[0]
        out_ref[...] = o * scale


def kernel(x, W_sw, b_sw, W1, b1, W2, b2):
    B, N, T, C = x.shape
    xf = x.reshape(-1, C)
    bsw2 = b_sw.reshape(1, N_EXPERTS)

    pos2, counts2 = pl.pallas_call(
        _router_body,
        grid=(NTOK // BT + 1,),
        in_specs=[
            pl.BlockSpec((BT, C), lambda i: (jnp.minimum(i, NTOK // BT - 1),
                                             0)),
            pl.BlockSpec((N_EXPERTS, C), lambda i: (0, 0)),
            pl.BlockSpec((1, N_EXPERTS), lambda i: (0, 0)),
        ],
        out_specs=[
            pl.BlockSpec((NTOK, 1), lambda i: (0, 0)),
            pl.BlockSpec((1, N_EXPERTS), lambda i: (0, 0)),
        ],
        out_shape=[
            jax.ShapeDtypeStruct((NTOK, 1), jnp.int32),
            jax.ShapeDtypeStruct((1, N_EXPERTS), jnp.int32),
        ],
        scratch_shapes=[
            pltpu.VMEM((1, N_EXPERTS), jnp.int32),
            pltpu.VMEM((NTOK, 1), jnp.int32),
            pltpu.VMEM((NTOK, 1), jnp.int32),
        ],
    )(xf, W_sw, bsw2)
    pos = pos2[:, 0]

    counts = counts2[0]
    blocks_e = (counts + BT - 1) // BT
    cum_blocks = jnp.cumsum(blocks_e)
    pad_off = cum_blocks - blocks_e
    k_blocks = cum_blocks[-1]

    bid = jnp.arange(NBLK, dtype=jnp.int32)
    eid_raw = jnp.sum((cum_blocks[None, :] <= bid[:, None]).astype(jnp.int32),
                      axis=1)
    last_eid = jnp.sum((cum_blocks <= (k_blocks - 1)).astype(jnp.int32))
    eids = jnp.where(bid < k_blocks, jnp.minimum(eid_raw, N_EXPERTS - 1),
                     last_eid).astype(jnp.int32)
    eids_ext = jnp.concatenate([eids, k_blocks[None].astype(jnp.int32)])

    xs = _make_sc_row_scatter(NTOK, C, NBLK * BT)(xf, pos)

    out_p = pl.pallas_call(
        _ffn_body,
        grid_spec=pltpu.PrefetchScalarGridSpec(
            num_scalar_prefetch=1,
            grid=(NBLK,),
            in_specs=[
                pl.BlockSpec((BT, C), lambda i, eid: (i, 0)),
                pl.BlockSpec((1, HIDDEN, C), lambda i, eid: (eid[i], 0, 0)),
                pl.BlockSpec((1, 1, HIDDEN), lambda i, eid: (eid[i], 0, 0)),
                pl.BlockSpec((1, C, HIDDEN), lambda i, eid: (eid[i], 0, 0)),
                pl.BlockSpec((1, 1, C), lambda i, eid: (eid[i], 0, 0)),
                pl.BlockSpec((N_EXPERTS, C), lambda i, eid: (0, 0)),
                pl.BlockSpec((1, N_EXPERTS), lambda i, eid: (0, 0)),
            ],
            out_specs=pl.BlockSpec((BT, C), lambda i, eid: (i, 0)),
        ),
        out_shape=jax.ShapeDtypeStruct((NBLK * BT, C), jnp.float32),
    )(eids_ext, xs, W1, b1.reshape(N_EXPERTS, 1, HIDDEN), W2,
      b2.reshape(N_EXPERTS, 1, C), W_sw, bsw2)

    out = _make_sc_row_gather(NBLK * BT, C, NTOK)(out_p, pos)
    return out.reshape(B, N, T, C)
